# Initial kernel scaffold; baseline (speedup 1.0000x reference)
import jax
import jax.numpy as jnp
from jax.experimental import pallas as pl


def kernel(input):
    raise NotImplementedError("write your pallas kernel here")


# speedup vs baseline: 1.1506x; 1.1506x over previous
import dataclasses

import jax
import jax.numpy as jnp
from jax import lax
from jax.experimental import pallas as pl
from jax.experimental.pallas import tpu as pltpu
from jax.experimental.pallas import tpu_sc as plsc

_CPARAMS = pltpu.CompilerParams()
if "needs_layout_passes" in pltpu.CompilerParams.__dataclass_fields__:
    _CPARAMS = dataclasses.replace(_CPARAMS, needs_layout_passes=False)

ROWS = 1024
COLS = 8192
NUM_OUT = 65536
L = 16
W = 128
GROUPS = 256
NC = 2
NS = 16
NW = NC * NS
ROWS_PER_W = ROWS // NW
CHUNK_GROUPS = 64
CHUNKS = GROUPS // CHUNK_GROUPS
CHUNK_OUT_ROWS = CHUNK_GROUPS * GROUPS // W
XROW_ROWS = COLS // W


def _sc_body(x_hbm, out_hbm, xrow_v, out_v):
    cid = lax.axis_index("c")
    sid = lax.axis_index("s")
    wid = sid * NC + cid

    @pl.loop(0, ROWS_PER_W)
    def _row_loop(r):
        row = wid * ROWS_PER_W + r
        pltpu.sync_copy(x_hbm.at[row], xrow_v)
        for chunk in range(CHUNKS):
            @pl.loop(0, CHUNK_GROUPS)
            def _group_loop(g):
                gg = chunk * CHUNK_GROUPS + g
                arow = gg // 4
                acol = (gg % 4) * 32
                arow_idx = jnp.full((L,), arow, jnp.int32)
                c_vec = xrow_v[arow, pl.ds(acol + L, L)]
                for i in range(L):
                    a_bcast = plsc.load_gather(
                        xrow_v, [arow_idx, jnp.full((L,), i, jnp.int32) + acol])
                    v = g * L + i
                    out_v[v // 8, pl.ds((v % 8) * L, L)] = c_vec + a_bcast
            pltpu.sync_copy(
                out_v,
                out_hbm.at[row, pl.ds(chunk * CHUNK_OUT_ROWS, CHUNK_OUT_ROWS)])


@jax.jit
def kernel(input):
    x3 = input.reshape(ROWS, XROW_ROWS, W)
    mesh = plsc.VectorSubcoreMesh(core_axis_name="c", subcore_axis_name="s")
    run = pl.kernel(
        _sc_body,
        out_type=jax.ShapeDtypeStruct((ROWS, NUM_OUT // W, W), jnp.float32),
        mesh=mesh,
        scratch_types=[
            pltpu.VMEM((XROW_ROWS, W), jnp.float32),
            pltpu.VMEM((CHUNK_OUT_ROWS, W), jnp.float32),
        ],
        compiler_params=_CPARAMS,
    )
    return run(x3).reshape(ROWS, NUM_OUT)


# speedup vs baseline: 1.3776x; 1.1972x over previous
import dataclasses

import jax
import jax.numpy as jnp
from jax import lax
from jax.experimental import pallas as pl
from jax.experimental.pallas import tpu as pltpu
from jax.experimental.pallas import tpu_sc as plsc

_CPARAMS = pltpu.CompilerParams()
if "needs_layout_passes" in pltpu.CompilerParams.__dataclass_fields__:
    _CPARAMS = dataclasses.replace(_CPARAMS, needs_layout_passes=False)

ROWS = 1024
COLS = 8192
NUM_OUT = 65536
L = 16
W = 128
NC = 2
NS = 16
NW = NC * NS
ROWS_PER_W = ROWS // NW
XROW_ROWS = COLS // W
OUT_ROWS = NUM_OUT // W
CHUNKS = 4
CHUNK_T = XROW_ROWS // CHUNKS
CHUNK_OUT_ROWS = OUT_ROWS // CHUNKS


def _sc_body(x_hbm, out_hbm, xrow0, xrow1, ob0, ob1, isem0, isem1, osem):
    cid = lax.axis_index("c")
    sid = lax.axis_index("s")
    wid = sid * NC + cid
    row0 = wid * ROWS_PER_W

    def compute_chunk(xrow_v, out_v, chunk):
        @pl.loop(0, CHUNK_T)
        def _t_loop(t):
            xr = chunk * CHUNK_T + t
            xr_idx = jnp.full((L,), 0, jnp.int32) + xr
            for q in range(4):
                c_vec = xrow_v[xr, pl.ds(q * 32 + L, L)]
                for i in range(L):
                    a_bcast = plsc.load_gather(
                        xrow_v,
                        [xr_idx, jnp.full((L,), q * 32 + i, jnp.int32)])
                    out_v[t * 8 + 2 * q + i // 8,
                          pl.ds((i % 8) * L, L)] = c_vec + a_bcast

    def do_row(r, xrow_v, xrow_next, isem, isem_next, first_row):
        row = row0 + r
        pltpu.make_async_copy(x_hbm.at[row], xrow_v, isem).wait()
        @pl.when(row + 1 < row0 + ROWS_PER_W)
        def _prefetch():
            pltpu.async_copy(x_hbm.at[row + 1], xrow_next, isem_next)

        for chunk in range(CHUNKS):
            ob = ob0 if chunk % 2 == 0 else ob1
            compute_chunk(xrow_v, ob, chunk)
            if not (first_row and chunk == 0):
                pltpu.make_async_copy(
                    ob, out_hbm.at[row, pl.ds(chunk * CHUNK_OUT_ROWS,
                                              CHUNK_OUT_ROWS)], osem).wait()
            pltpu.async_copy(
                ob, out_hbm.at[row, pl.ds(chunk * CHUNK_OUT_ROWS,
                                          CHUNK_OUT_ROWS)], osem)

    pltpu.async_copy(x_hbm.at[row0], xrow0, isem0)
    do_row(0, xrow0, xrow1, isem0, isem1, True)
    do_row(1, xrow1, xrow0, isem1, isem0, False)

    @pl.loop(1, ROWS_PER_W // 2)
    def _row_pair(rp):
        do_row(2 * rp, xrow0, xrow1, isem0, isem1, False)
        do_row(2 * rp + 1, xrow1, xrow0, isem1, isem0, False)

    pltpu.make_async_copy(
        ob1, out_hbm.at[row0, pl.ds(0, CHUNK_OUT_ROWS)], osem).wait()


@jax.jit
def kernel(input):
    x3 = input.reshape(ROWS, XROW_ROWS, W)
    mesh = plsc.VectorSubcoreMesh(core_axis_name="c", subcore_axis_name="s")
    run = pl.kernel(
        _sc_body,
        out_type=jax.ShapeDtypeStruct((ROWS, OUT_ROWS, W), jnp.float32),
        mesh=mesh,
        scratch_types=[
            pltpu.VMEM((XROW_ROWS, W), jnp.float32),
            pltpu.VMEM((XROW_ROWS, W), jnp.float32),
            pltpu.VMEM((CHUNK_OUT_ROWS, W), jnp.float32),
            pltpu.VMEM((CHUNK_OUT_ROWS, W), jnp.float32),
            pltpu.SemaphoreType.DMA,
            pltpu.SemaphoreType.DMA,
            pltpu.SemaphoreType.DMA,
        ],
        compiler_params=_CPARAMS,
    )
    return run(x3).reshape(ROWS, NUM_OUT)


# speedup vs baseline: 8.3124x; 6.0340x over previous
import dataclasses

import jax
import jax.numpy as jnp
from jax import lax
from jax.experimental import pallas as pl
from jax.experimental.pallas import tpu as pltpu
from jax.experimental.pallas import tpu_sc as plsc

_CPARAMS = pltpu.CompilerParams()
if "needs_layout_passes" in pltpu.CompilerParams.__dataclass_fields__:
    _CPARAMS = dataclasses.replace(_CPARAMS, needs_layout_passes=False)

ROWS = 1024
COLS = 8192
NUM_OUT = 65536
L = 16
NC = 2
NS = 16
NW = NC * NS
ROWS_PER_W = ROWS // NW
GROUPS = 256
CHUNKS = 4
CHUNK_G = GROUPS // CHUNKS
CHUNK_WORDS = CHUNK_G * GROUPS
GQ = 4


def _sc_body(x_hbm, out_hbm, xrow0, xrow1, ob0, ob1, isem0, isem1, osem):
    cid = lax.axis_index("c")
    sid = lax.axis_index("s")
    wid = sid * NC + cid
    row0 = wid * ROWS_PER_W

    idx_splats = [jnp.full((L, 1), i, jnp.int32) for i in range(L)]
    _dnums = lax.GatherDimensionNumbers(
        offset_dims=(), collapsed_slice_dims=(0,), start_index_map=(0,))

    def _bcast_lane(vec, i):
        return lax.gather(vec, idx_splats[i], _dnums, (1,),
                          mode=lax.GatherScatterMode.PROMISE_IN_BOUNDS)

    def compute_chunk(xrow_v, out_v, chunk):
        @pl.loop(0, CHUNK_G // GQ)
        def _g_loop(gq):
            g_base = gq * GQ
            for q in range(GQ):
                g = g_base + q
                src = (chunk * CHUNK_G + g) * 32
                a_vec = xrow_v[pl.ds(src, L)]
                c_vec = xrow_v[pl.ds(src + L, L)]
                for i in range(L):
                    a_bcast = _bcast_lane(a_vec, i)
                    out_v[pl.ds(g * 256 + i * L, L)] = c_vec + a_bcast

    def do_row(r, xrow_v, xrow_next, isem, isem_next, first_row):
        row = row0 + r
        pltpu.make_async_copy(x_hbm.at[row], xrow_v, isem).wait()
        @pl.when(row + 1 < row0 + ROWS_PER_W)
        def _prefetch():
            pltpu.async_copy(x_hbm.at[row + 1], xrow_next, isem_next)

        for chunk in range(CHUNKS):
            ob = ob0 if chunk % 2 == 0 else ob1
            compute_chunk(xrow_v, ob, chunk)
            dst = out_hbm.at[row, pl.ds(chunk * CHUNK_WORDS, CHUNK_WORDS)]
            if not (first_row and chunk == 0):
                pltpu.make_async_copy(ob, dst, osem).wait()
            pltpu.async_copy(ob, dst, osem)

    pltpu.async_copy(x_hbm.at[row0], xrow0, isem0)
    do_row(0, xrow0, xrow1, isem0, isem1, True)
    do_row(1, xrow1, xrow0, isem1, isem0, False)

    @pl.loop(1, ROWS_PER_W // 2)
    def _row_pair(rp):
        do_row(2 * rp, xrow0, xrow1, isem0, isem1, False)
        do_row(2 * rp + 1, xrow1, xrow0, isem1, isem0, False)

    pltpu.make_async_copy(
        ob1, out_hbm.at[row0, pl.ds(0, CHUNK_WORDS)], osem).wait()


@jax.jit
def kernel(input):
    mesh = plsc.VectorSubcoreMesh(core_axis_name="c", subcore_axis_name="s")
    run = pl.kernel(
        _sc_body,
        out_type=jax.ShapeDtypeStruct((ROWS, NUM_OUT), jnp.float32),
        mesh=mesh,
        scratch_types=[
            pltpu.VMEM((COLS,), jnp.float32),
            pltpu.VMEM((COLS,), jnp.float32),
            pltpu.VMEM((CHUNK_WORDS,), jnp.float32),
            pltpu.VMEM((CHUNK_WORDS,), jnp.float32),
            pltpu.SemaphoreType.DMA,
            pltpu.SemaphoreType.DMA,
            pltpu.SemaphoreType.DMA,
        ],
        compiler_params=_CPARAMS,
    )
    return run(input)


# speedup vs baseline: 8.4229x; 1.0133x over previous
import dataclasses

import jax
import jax.numpy as jnp
from jax import lax
from jax.experimental import pallas as pl
from jax.experimental.pallas import tpu as pltpu
from jax.experimental.pallas import tpu_sc as plsc

_CPARAMS = pltpu.CompilerParams()
if "needs_layout_passes" in pltpu.CompilerParams.__dataclass_fields__:
    _CPARAMS = dataclasses.replace(_CPARAMS, needs_layout_passes=False)

ROWS = 1024
COLS = 8192
NUM_OUT = 65536
L = 16
NC = 2
NS = 16
NW = NC * NS
ROWS_PER_W = ROWS // NW
GROUPS = 256
CHUNKS = 2
CHUNK_G = GROUPS // CHUNKS
CHUNK_WORDS = CHUNK_G * GROUPS
GQ = 4


def _sc_body(x_hbm, out_hbm, xrow0, xrow1, ob0, ob1, isem0, isem1, osem):
    cid = lax.axis_index("c")
    sid = lax.axis_index("s")
    wid = sid * NC + cid
    row0 = wid * ROWS_PER_W

    idx_splats = [jnp.full((L, 1), i, jnp.int32) for i in range(L)]
    _dnums = lax.GatherDimensionNumbers(
        offset_dims=(), collapsed_slice_dims=(0,), start_index_map=(0,))

    def _bcast_lane(vec, i):
        return lax.gather(vec, idx_splats[i], _dnums, (1,),
                          mode=lax.GatherScatterMode.PROMISE_IN_BOUNDS)

    def compute_chunk(xrow_v, out_v, chunk):
        @pl.loop(0, CHUNK_G // GQ)
        def _g_loop(gq):
            g_base = gq * GQ
            for q in range(GQ):
                g = g_base + q
                src = (chunk * CHUNK_G + g) * 32
                a_vec = xrow_v[pl.ds(src, L)]
                c_vec = xrow_v[pl.ds(src + L, L)]
                for i in range(L):
                    a_bcast = _bcast_lane(a_vec, i)
                    out_v[pl.ds(g * 256 + i * L, L)] = c_vec + a_bcast

    def do_row(r, xrow_v, xrow_next, isem, isem_next, first_row):
        row = row0 + r
        pltpu.make_async_copy(x_hbm.at[row], xrow_v, isem).wait()
        @pl.when(row + 1 < row0 + ROWS_PER_W)
        def _prefetch():
            pltpu.async_copy(x_hbm.at[row + 1], xrow_next, isem_next)

        for chunk in range(CHUNKS):
            ob = ob0 if chunk % 2 == 0 else ob1
            compute_chunk(xrow_v, ob, chunk)
            dst = out_hbm.at[row, pl.ds(chunk * CHUNK_WORDS, CHUNK_WORDS)]
            if not (first_row and chunk == 0):
                pltpu.make_async_copy(ob, dst, osem).wait()
            pltpu.async_copy(ob, dst, osem)

    pltpu.async_copy(x_hbm.at[row0], xrow0, isem0)
    do_row(0, xrow0, xrow1, isem0, isem1, True)
    do_row(1, xrow1, xrow0, isem1, isem0, False)

    @pl.loop(1, ROWS_PER_W // 2)
    def _row_pair(rp):
        do_row(2 * rp, xrow0, xrow1, isem0, isem1, False)
        do_row(2 * rp + 1, xrow1, xrow0, isem1, isem0, False)

    pltpu.make_async_copy(
        ob1, out_hbm.at[row0, pl.ds(0, CHUNK_WORDS)], osem).wait()


@jax.jit
def kernel(input):
    mesh = plsc.VectorSubcoreMesh(core_axis_name="c", subcore_axis_name="s")
    run = pl.kernel(
        _sc_body,
        out_type=jax.ShapeDtypeStruct((ROWS, NUM_OUT), jnp.float32),
        mesh=mesh,
        scratch_types=[
            pltpu.VMEM((COLS,), jnp.float32),
            pltpu.VMEM((COLS,), jnp.float32),
            pltpu.VMEM((CHUNK_WORDS,), jnp.float32),
            pltpu.VMEM((CHUNK_WORDS,), jnp.float32),
            pltpu.SemaphoreType.DMA,
            pltpu.SemaphoreType.DMA,
            pltpu.SemaphoreType.DMA,
        ],
        compiler_params=_CPARAMS,
    )
    return run(input)
